# baseline (speedup 1.0000x reference)
import functools

import jax
import jax.numpy as jnp
from jax.experimental import pallas as pl
from jax.experimental.pallas import tpu as pltpu

_ALPHA = 32.0
_DN_T = (((1,), (1,)), ((), ()))
_DN_N = (((1,), (0,)), ((), ()))


def _body(eids_ref, x_ref, w_ref, la_ref, lbt_ref, b_ref, o_ref,
          *, bt: int, rank: int, er: int, scaling: float):
    xb = x_ref[...].astype(jnp.bfloat16)
    inter = jax.lax.dot_general(xb, la_ref[...], _DN_T,
                                preferred_element_type=jnp.float32)
    eids = eids_ref[0, 0, :]
    col_expert = jax.lax.broadcasted_iota(jnp.int32, (bt, er), 1) // rank
    mask = (col_expert == eids[:, None]).astype(inter.dtype)
    inter_m = (inter * mask).astype(jnp.bfloat16)
    base = jax.lax.dot_general(xb, w_ref[...], _DN_T,
                               preferred_element_type=jnp.float32)
    delta = jax.lax.dot_general(inter_m, lbt_ref[...], _DN_N,
                                preferred_element_type=jnp.float32)
    o_ref[...] = base + b_ref[...] + delta * scaling


def kernel(x, expert_ids, W, b, lora_a, lora_b):
    num_tokens, d_in = x.shape
    d_out = W.shape[0]
    num_experts, rank, _ = lora_a.shape
    er = num_experts * rank
    scaling = _ALPHA / float(rank)

    cdt = jnp.bfloat16
    wc = W.astype(cdt)
    la = lora_a.reshape(er, d_in).astype(cdt)
    lbt = (lora_b.transpose(0, 2, 1)
           .reshape(er, d_out).astype(cdt))
    b2 = b.reshape(1, d_out)

    bt = 256
    nt = num_tokens // bt
    eids3 = expert_ids.astype(jnp.int32).reshape(nt, 1, bt)

    out = pl.pallas_call(
        functools.partial(_body, bt=bt, rank=rank, er=er, scaling=scaling),
        grid=(nt,),
        in_specs=[
            pl.BlockSpec((1, 1, bt), lambda i: (i, 0, 0)),
            pl.BlockSpec((bt, d_in), lambda i: (i, 0)),
            pl.BlockSpec((d_out, d_in), lambda i: (0, 0)),
            pl.BlockSpec((er, d_in), lambda i: (0, 0)),
            pl.BlockSpec((er, d_out), lambda i: (0, 0)),
            pl.BlockSpec((1, d_out), lambda i: (0, 0)),
        ],
        out_specs=pl.BlockSpec((bt, d_out), lambda i: (i, 0)),
        out_shape=jax.ShapeDtypeStruct((num_tokens, d_out), jnp.float32),
        compiler_params=pltpu.CompilerParams(
            dimension_semantics=("parallel",),
        ),
    )(eids3, x, wc, la, lbt, b2)
    return out


# speedup vs baseline: 1.0521x; 1.0521x over previous
import functools

import jax
import jax.numpy as jnp
from jax.experimental import pallas as pl
from jax.experimental.pallas import tpu as pltpu

_ALPHA = 32.0
_DN_T = (((1,), (1,)), ((), ()))
_DN_N = (((1,), (0,)), ((), ()))


def _body(eids_ref, x_ref, w_ref, la_ref, lbt_ref, b_ref, o_ref,
          *, bt: int, rank: int, er: int, scaling: float):
    xb = x_ref[...].astype(jnp.bfloat16)
    inter = jax.lax.dot_general(xb, la_ref[...], _DN_T,
                                preferred_element_type=jnp.float32)
    eids = eids_ref[0, 0, :]
    col_expert = jax.lax.broadcasted_iota(jnp.int32, (bt, er), 1) // rank
    mask = (col_expert == eids[:, None]).astype(inter.dtype)
    inter_m = (inter * mask).astype(jnp.bfloat16)
    base = jax.lax.dot_general(xb, w_ref[...], _DN_T,
                               preferred_element_type=jnp.float32)
    delta = jax.lax.dot_general(inter_m, lbt_ref[...], _DN_N,
                                preferred_element_type=jnp.float32)
    o_ref[...] = base + b_ref[...] + delta * scaling


def kernel(x, expert_ids, W, b, lora_a, lora_b):
    num_tokens, d_in = x.shape
    d_out = W.shape[0]
    num_experts, rank, _ = lora_a.shape
    er = num_experts * rank
    scaling = _ALPHA / float(rank)

    cdt = jnp.bfloat16
    wc = W.astype(cdt)
    la = lora_a.reshape(er, d_in).astype(cdt)
    lbt = (lora_b.transpose(0, 2, 1)
           .reshape(er, d_out).astype(cdt))
    b2 = b.reshape(1, d_out)

    bt = 1024
    nt = num_tokens // bt
    eids3 = expert_ids.astype(jnp.int32).reshape(nt, 1, bt)

    out = pl.pallas_call(
        functools.partial(_body, bt=bt, rank=rank, er=er, scaling=scaling),
        grid=(nt,),
        in_specs=[
            pl.BlockSpec((1, 1, bt), lambda i: (i, 0, 0)),
            pl.BlockSpec((bt, d_in), lambda i: (i, 0)),
            pl.BlockSpec((d_out, d_in), lambda i: (0, 0)),
            pl.BlockSpec((er, d_in), lambda i: (0, 0)),
            pl.BlockSpec((er, d_out), lambda i: (0, 0)),
            pl.BlockSpec((1, d_out), lambda i: (0, 0)),
        ],
        out_specs=pl.BlockSpec((bt, d_out), lambda i: (i, 0)),
        out_shape=jax.ShapeDtypeStruct((num_tokens, d_out), jnp.float32),
        compiler_params=pltpu.CompilerParams(
            dimension_semantics=("parallel",),
        ),
    )(eids3, x, wc, la, lbt, b2)
    return out
